# initial kernel scaffold (unmeasured)
import jax
import jax.numpy as jnp
from jax import lax
from jax.experimental import pallas as pl
from jax.experimental.pallas import tpu as pltpu

B, SQ, H, D = 4, 256, 16, 64
HD = H * D
ROWS = B * SQ
SCALE = D ** -0.5


def _comm_body(k_ref, v_ref, kr_ref, vr_ref, ksend, vsend, send_sems, recv_sems):
    my_x = lax.axis_index("x")
    my_y = lax.axis_index("y")
    nbr = (my_x, 1 - my_y)

    ksend[...] = k_ref[...].astype(jnp.bfloat16)
    vsend[...] = v_ref[...].astype(jnp.bfloat16)

    barrier = pltpu.get_barrier_semaphore()
    pl.semaphore_signal(barrier, inc=1, device_id=nbr,
                        device_id_type=pl.DeviceIdType.MESH)
    pl.semaphore_wait(barrier, 1)

    rk = pltpu.make_async_remote_copy(
        src_ref=ksend, dst_ref=kr_ref,
        send_sem=send_sems.at[0], recv_sem=recv_sems.at[0],
        device_id=nbr, device_id_type=pl.DeviceIdType.MESH)
    rv = pltpu.make_async_remote_copy(
        src_ref=vsend, dst_ref=vr_ref,
        send_sem=send_sems.at[1], recv_sem=recv_sems.at[1],
        device_id=nbr, device_id_type=pl.DeviceIdType.MESH)
    rk.start()
    rv.start()
    rk.wait()
    rv.wait()


def _exchange(k2d, v2d):
    return pl.pallas_call(
        _comm_body,
        out_shape=(
            jax.ShapeDtypeStruct((ROWS, HD), jnp.bfloat16),
            jax.ShapeDtypeStruct((ROWS, HD), jnp.bfloat16),
        ),
        in_specs=[pl.BlockSpec(memory_space=pltpu.VMEM)] * 2,
        out_specs=(pl.BlockSpec(memory_space=pltpu.VMEM),) * 2,
        scratch_shapes=[
            pltpu.VMEM((ROWS, HD), jnp.bfloat16),
            pltpu.VMEM((ROWS, HD), jnp.bfloat16),
            pltpu.SemaphoreType.DMA((2,)),
            pltpu.SemaphoreType.DMA((2,)),
        ],
        compiler_params=pltpu.CompilerParams(collective_id=0),
    )(k2d, v2d)


def _attn_body(q_ref, kl_ref, vl_ref, kr_ref, vr_ref, o_ref):
    q = q_ref[0].astype(jnp.bfloat16)
    kl = kl_ref[0].astype(jnp.bfloat16)
    vl = vl_ref[0].astype(jnp.bfloat16)
    kr = kr_ref[0]
    vr = vr_ref[0]

    nt = (((1,), (1,)), ((), ()))
    nn = (((1,), (0,)), ((), ()))
    s1 = lax.dot_general(q, kl, nt, preferred_element_type=jnp.float32) * SCALE
    s2 = lax.dot_general(q, kr, nt, preferred_element_type=jnp.float32) * SCALE
    m = jnp.maximum(jnp.max(s1, axis=1, keepdims=True),
                    jnp.max(s2, axis=1, keepdims=True))
    p1 = jnp.exp(s1 - m)
    p2 = jnp.exp(s2 - m)
    denom = jnp.sum(p1, axis=1, keepdims=True) + jnp.sum(p2, axis=1, keepdims=True)
    o1 = lax.dot_general(p1.astype(jnp.bfloat16), vl, nn,
                         preferred_element_type=jnp.float32)
    o2 = lax.dot_general(p2.astype(jnp.bfloat16), vr, nn,
                         preferred_element_type=jnp.float32)
    o_ref[0] = (o1 + o2) / denom


def kernel(Q, K, V):
    q3 = Q.reshape(B, SQ, HD)
    k3 = K.reshape(B, SQ, HD)
    v3 = V.reshape(B, SQ, HD)

    k_rem, v_rem = _exchange(k3.reshape(ROWS, HD), v3.reshape(ROWS, HD))
    k_rem = k_rem.reshape(B, SQ, HD)
    v_rem = v_rem.reshape(B, SQ, HD)

    blk = lambda: pl.BlockSpec((1, SQ, D), lambda b, h: (b, 0, h))
    out = pl.pallas_call(
        _attn_body,
        grid=(B, H),
        in_specs=[blk() for _ in range(5)],
        out_specs=blk(),
        out_shape=jax.ShapeDtypeStruct((B, SQ, HD), jnp.float32),
        compiler_params=pltpu.CompilerParams(
            dimension_semantics=("arbitrary", "arbitrary")),
    )(q3, k3, v3, k_rem, v_rem)
    return out.reshape(B, SQ, H, D)


# baseline (device time: 111916 ns/iter reference)
import jax
import jax.numpy as jnp
from jax import lax
from jax.experimental import pallas as pl
from jax.experimental.pallas import tpu as pltpu

B, SQ, H, D = 4, 256, 16, 64
HD = H * D
ROWS = B * SQ
SCALE = D ** -0.5


def _comm_body(k_ref, v_ref, kr_ref, vr_ref, ksend, vsend, send_sems, recv_sems):
    my_x = lax.axis_index("x")
    my_y = lax.axis_index("y")
    nbr = (my_x, 1 - my_y)

    ksend[...] = k_ref[...].astype(jnp.bfloat16)
    vsend[...] = v_ref[...].astype(jnp.bfloat16)

    barrier = pltpu.get_barrier_semaphore()
    pl.semaphore_signal(barrier, inc=1, device_id=nbr,
                        device_id_type=pl.DeviceIdType.MESH)
    pl.semaphore_wait(barrier, 1)

    rk = pltpu.make_async_remote_copy(
        src_ref=ksend, dst_ref=kr_ref,
        send_sem=send_sems.at[0], recv_sem=recv_sems.at[0],
        device_id=nbr, device_id_type=pl.DeviceIdType.MESH)
    rv = pltpu.make_async_remote_copy(
        src_ref=vsend, dst_ref=vr_ref,
        send_sem=send_sems.at[1], recv_sem=recv_sems.at[1],
        device_id=nbr, device_id_type=pl.DeviceIdType.MESH)
    rk.start()
    rv.start()
    rk.wait()
    rv.wait()


def _exchange(k2d, v2d):
    return pl.pallas_call(
        _comm_body,
        out_shape=(
            jax.ShapeDtypeStruct((ROWS, HD), jnp.bfloat16),
            jax.ShapeDtypeStruct((ROWS, HD), jnp.bfloat16),
        ),
        in_specs=[pl.BlockSpec(memory_space=pltpu.VMEM)] * 2,
        out_specs=(pl.BlockSpec(memory_space=pltpu.VMEM),) * 2,
        scratch_shapes=[
            pltpu.VMEM((ROWS, HD), jnp.bfloat16),
            pltpu.VMEM((ROWS, HD), jnp.bfloat16),
            pltpu.SemaphoreType.DMA((2,)),
            pltpu.SemaphoreType.DMA((2,)),
        ],
        compiler_params=pltpu.CompilerParams(collective_id=0),
    )(k2d, v2d)


def _one_head(q, kl, vl, kr, vr):
    nt = (((1,), (1,)), ((), ()))
    nn = (((1,), (0,)), ((), ()))
    s1 = lax.dot_general(q, kl, nt, preferred_element_type=jnp.float32) * SCALE
    s2 = lax.dot_general(q, kr, nt, preferred_element_type=jnp.float32) * SCALE
    m = jnp.maximum(jnp.max(s1, axis=1, keepdims=True),
                    jnp.max(s2, axis=1, keepdims=True))
    p1 = jnp.exp(s1 - m)
    p2 = jnp.exp(s2 - m)
    denom = jnp.sum(p1, axis=1, keepdims=True) + jnp.sum(p2, axis=1, keepdims=True)
    o1 = lax.dot_general(p1.astype(jnp.bfloat16), vl, nn,
                         preferred_element_type=jnp.float32)
    o2 = lax.dot_general(p2.astype(jnp.bfloat16), vr, nn,
                         preferred_element_type=jnp.float32)
    return (o1 + o2) / denom


def _attn_body(q_ref, kl_ref, vl_ref, kr_ref, vr_ref, o_ref):
    q = q_ref[0].astype(jnp.bfloat16)
    kl = kl_ref[0].astype(jnp.bfloat16)
    vl = vl_ref[0].astype(jnp.bfloat16)
    kr = kr_ref[0]
    vr = vr_ref[0]
    outs = [
        _one_head(q[:, s], kl[:, s], vl[:, s], kr[:, s], vr[:, s])
        for s in (slice(0, D), slice(D, 2 * D))
    ]
    o_ref[0] = jnp.concatenate(outs, axis=1)


def kernel(Q, K, V):
    q3 = Q.reshape(B, SQ, HD)
    k3 = K.reshape(B, SQ, HD)
    v3 = V.reshape(B, SQ, HD)

    k_rem, v_rem = _exchange(k3.reshape(ROWS, HD), v3.reshape(ROWS, HD))
    k_rem = k_rem.reshape(B, SQ, HD)
    v_rem = v_rem.reshape(B, SQ, HD)

    blk = lambda: pl.BlockSpec((1, SQ, 2 * D), lambda b, h: (b, 0, h))
    out = pl.pallas_call(
        _attn_body,
        grid=(B, H // 2),
        in_specs=[blk() for _ in range(5)],
        out_specs=blk(),
        out_shape=jax.ShapeDtypeStruct((B, SQ, HD), jnp.float32),
        compiler_params=pltpu.CompilerParams(
            dimension_semantics=("arbitrary", "arbitrary")),
    )(q3, k3, v3, k_rem, v_rem)
    return out.reshape(B, SQ, H, D)


# device time: 98431 ns/iter; 1.1370x vs baseline; 1.1370x over previous
import jax
import jax.numpy as jnp
from jax import lax
from jax.experimental import pallas as pl
from jax.experimental.pallas import tpu as pltpu

B, SQ, H, D = 4, 256, 16, 64
HD = H * D
ROWS = B * SQ
SCALE = D ** -0.5


C = 8
CH = ROWS // C


def _comm_body(k_ref, v_ref, kr_ref, vr_ref, ksend, vsend, krbuf, vrbuf,
               s1, r1, s2, r2, ybar, xbar):
    my_x = lax.axis_index("x")
    my_y = lax.axis_index("y")
    ynbr = (my_x, 1 - my_y)
    xnbr = (1 - my_x, my_y)

    ksend[...] = k_ref[...].astype(jnp.bfloat16)
    vsend[...] = v_ref[...].astype(jnp.bfloat16)

    pl.semaphore_signal(ybar, inc=1, device_id=ynbr,
                        device_id_type=pl.DeviceIdType.MESH)
    pl.semaphore_signal(xbar, inc=1, device_id=xnbr,
                        device_id_type=pl.DeviceIdType.MESH)
    pl.semaphore_wait(ybar, 1)
    pl.semaphore_wait(xbar, 1)

    def run(send_buf, recv1, recv2, out1, out2):
        mk = lambda **kw: pltpu.make_async_remote_copy(
            device_id_type=pl.DeviceIdType.MESH, **kw)
        ch = lambda ref, i: ref.at[pl.ds(i * CH, CH)]
        p1 = [mk(src_ref=ch(send_buf, i), dst_ref=ch(recv1, i),
                 send_sem=s1.at[i], recv_sem=r1.at[i], device_id=ynbr)
              for i in range(C)]
        fwd = [mk(src_ref=ch(recv1, i), dst_ref=ch(recv1, i),
                  send_sem=s2.at[i], recv_sem=r2.at[i], device_id=xnbr)
               for i in range(C)]
        p2w = [mk(src_ref=ch(send_buf, i), dst_ref=ch(recv2, i),
                  send_sem=s1.at[i], recv_sem=r2.at[i], device_id=xnbr)
               for i in range(C)]
        for d in p1:
            d.start()
        for i in range(C):
            p1[i].wait_recv()
            fwd[i].start()
            out1[pl.ds(i * CH, CH), :] = recv1[pl.ds(i * CH, CH), :]
        for i in range(C):
            p2w[i].wait_recv()
            out2[pl.ds(i * CH, CH), :] = recv2[pl.ds(i * CH, CH), :]
        for i in range(C):
            p1[i].wait_send()
            fwd[i].wait_send()

    @pl.when(my_x == 0)
    def _():
        run(ksend, krbuf, vrbuf, kr_ref, vr_ref)

    @pl.when(my_x == 1)
    def _():
        run(vsend, vrbuf, krbuf, vr_ref, kr_ref)


def _exchange(k2d, v2d):
    return pl.pallas_call(
        _comm_body,
        out_shape=(
            jax.ShapeDtypeStruct((ROWS, HD), jnp.bfloat16),
            jax.ShapeDtypeStruct((ROWS, HD), jnp.bfloat16),
        ),
        in_specs=[pl.BlockSpec(memory_space=pltpu.VMEM)] * 2,
        out_specs=(pl.BlockSpec(memory_space=pltpu.VMEM),) * 2,
        scratch_shapes=[
            pltpu.VMEM((ROWS, HD), jnp.bfloat16),
            pltpu.VMEM((ROWS, HD), jnp.bfloat16),
            pltpu.VMEM((ROWS, HD), jnp.bfloat16),
            pltpu.VMEM((ROWS, HD), jnp.bfloat16),
            pltpu.SemaphoreType.DMA((C,)),
            pltpu.SemaphoreType.DMA((C,)),
            pltpu.SemaphoreType.DMA((C,)),
            pltpu.SemaphoreType.DMA((C,)),
            pltpu.SemaphoreType.REGULAR,
            pltpu.SemaphoreType.REGULAR,
        ],
    )(k2d, v2d)


def _one_head(q, kl, vl, kr, vr):
    nt = (((1,), (1,)), ((), ()))
    nn = (((1,), (0,)), ((), ()))
    s1 = lax.dot_general(q, kl, nt, preferred_element_type=jnp.float32) * SCALE
    s2 = lax.dot_general(q, kr, nt, preferred_element_type=jnp.float32) * SCALE
    m = jnp.maximum(jnp.max(s1, axis=1, keepdims=True),
                    jnp.max(s2, axis=1, keepdims=True))
    p1 = jnp.exp(s1 - m)
    p2 = jnp.exp(s2 - m)
    denom = jnp.sum(p1, axis=1, keepdims=True) + jnp.sum(p2, axis=1, keepdims=True)
    o1 = lax.dot_general(p1.astype(jnp.bfloat16), vl, nn,
                         preferred_element_type=jnp.float32)
    o2 = lax.dot_general(p2.astype(jnp.bfloat16), vr, nn,
                         preferred_element_type=jnp.float32)
    return (o1 + o2) / denom


def _attn_body(q_ref, kl_ref, vl_ref, kr_ref, vr_ref, o_ref):
    q = q_ref[0].astype(jnp.bfloat16)
    kl = kl_ref[0].astype(jnp.bfloat16)
    vl = vl_ref[0].astype(jnp.bfloat16)
    kr = kr_ref[0]
    vr = vr_ref[0]
    outs = [
        _one_head(q[:, s], kl[:, s], vl[:, s], kr[:, s], vr[:, s])
        for s in (slice(0, D), slice(D, 2 * D))
    ]
    o_ref[0] = jnp.concatenate(outs, axis=1)


def kernel(Q, K, V):
    q3 = Q.reshape(B, SQ, HD)
    k3 = K.reshape(B, SQ, HD)
    v3 = V.reshape(B, SQ, HD)

    k_rem, v_rem = _exchange(k3.reshape(ROWS, HD), v3.reshape(ROWS, HD))
    k_rem = k_rem.reshape(B, SQ, HD)
    v_rem = v_rem.reshape(B, SQ, HD)

    blk = lambda: pl.BlockSpec((1, SQ, 2 * D), lambda b, h: (b, 0, h))
    out = pl.pallas_call(
        _attn_body,
        grid=(B, H // 2),
        in_specs=[blk() for _ in range(5)],
        out_specs=blk(),
        out_shape=jax.ShapeDtypeStruct((B, SQ, HD), jnp.float32),
        compiler_params=pltpu.CompilerParams(
            dimension_semantics=("arbitrary", "arbitrary")),
    )(q3, k3, v3, k_rem, v_rem)
    return out.reshape(B, SQ, H, D)
